# Initial kernel scaffold; baseline (speedup 1.0000x reference)
import jax
import jax.numpy as jnp
from jax.experimental import pallas as pl


def kernel(sup_x, y, assign_index, assign_weight, anchor_links, anchor_weight, num_nodes, W1, W2):
    raise NotImplementedError("write your pallas kernel here")


# speedup vs baseline: 5.8036x; 5.8036x over previous
import functools

import jax
import jax.numpy as jnp
from jax import lax
from jax.experimental import pallas as pl
from jax.experimental.pallas import tpu as pltpu
from jax.experimental.pallas import tpu_sc as plsc

D = 128
N_NODES = 10000
N_EDGES = 320000

NC = 2
NS = 16
NW = NC * NS

EPT = N_EDGES // NW
CH = 80
NCHUNK = EPT // CH
NPAD = 10240
ROWS_PT = NPAD // NS


def _lane_bcast(vec, j):
    idx = jnp.full((16, 1), j, dtype=jnp.int32)
    return lax.gather(
        vec, idx,
        lax.GatherDimensionNumbers(
            offset_dims=(), collapsed_slice_dims=(0,), start_index_map=(0,)),
        (1,),
        mode=lax.GatherScatterMode.PROMISE_IN_BOUNDS)


def _sc_body(t1, t2, srca, dsta, wa, srcb, dstb, wb, zeros_hbm, out,
             acc, src_c, dst_c, rows, src_big, dst_big, w_big, sem):
    cid = lax.axis_index("c")
    sid = lax.axis_index("s")
    wid = sid * NC + cid

    pltpu.sync_copy(zeros_hbm.at[pl.ds(sid * ROWS_PT, ROWS_PT)],
                    acc.at[pl.ds(sid * ROWS_PT, ROWS_PT)])
    plsc.subcore_barrier()

    for (src_h, dst_h, w_h, tab) in ((srca, dsta, wa, t1), (srcb, dstb, wb, t2)):
        base = wid * EPT
        pltpu.sync_copy(src_h.at[pl.ds(base, EPT)], src_big)
        pltpu.sync_copy(dst_h.at[pl.ds(base, EPT)], dst_big)
        pltpu.sync_copy(w_h.at[pl.ds(base, EPT)], w_big)

        def chunk_body(c, carry, src_h=src_h, dst_h=dst_h, w_h=w_h, tab=tab):
            off = c * CH
            for k in range(CH // 16):
                src_c[pl.ds(16 * k, 16)] = src_big[pl.ds(off + 16 * k, 16)]
                dst_c[pl.ds(16 * k, 16)] = dst_big[pl.ds(off + 16 * k, 16)]
            pltpu.async_copy(tab.at[src_c], rows, sem).wait()

            def scale_grp(g, carry2):
                wv = w_big[pl.ds(off + 16 * g, 16)]
                for j in range(16):
                    wbc = _lane_bcast(wv, j)
                    e = 16 * g + j
                    for v in range(8):
                        rows[e, pl.ds(16 * v, 16)] = rows[e, pl.ds(16 * v, 16)] * wbc
                return carry2
            lax.fori_loop(0, CH // 16, scale_grp, 0)

            pltpu.sync_copy(rows, acc.at[dst_c], add=True)
            return carry
        lax.fori_loop(0, NCHUNK, chunk_body, 0)

    plsc.subcore_barrier()
    pltpu.sync_copy(acc.at[pl.ds(sid * ROWS_PT, ROWS_PT)],
                    out.at[cid, pl.ds(sid * ROWS_PT, ROWS_PT)])


def _scatter_partials(t1, t2, srca, dsta, wa, srcb, dstb, wb, zeros_hbm):
    mesh = plsc.VectorSubcoreMesh(core_axis_name="c", subcore_axis_name="s")
    return pl.kernel(
        _sc_body,
        mesh=mesh,
        out_type=jax.ShapeDtypeStruct((NC, NPAD, D), jnp.float32),
        scratch_types=[
            pltpu.VMEM_SHARED((NPAD, D), jnp.float32),
            pltpu.VMEM((CH,), jnp.int32),
            pltpu.VMEM((CH,), jnp.int32),
            pltpu.VMEM((CH, D), jnp.float32),
            pltpu.VMEM((EPT,), jnp.int32),
            pltpu.VMEM((EPT,), jnp.int32),
            pltpu.VMEM((EPT,), jnp.float32),
            pltpu.SemaphoreType.DMA,
        ],
    )(t1, t2, srca, dsta, wa, srcb, dstb, wb, zeros_hbm)


BM = 2000


def _mm_body(x_ref, y_ref, w1_ref, w2_ref, o1_ref, o2_ref):
    dn = (((1,), (1,)), ((), ()))
    o1_ref[...] = lax.dot_general(x_ref[...], w1_ref[...], dn,
                                  preferred_element_type=jnp.float32)
    o2_ref[...] = lax.dot_general(y_ref[...], w2_ref[...], dn,
                                  preferred_element_type=jnp.float32)


def _add_body(a_ref, b_ref, o_ref):
    o_ref[...] = a_ref[...] + b_ref[...]


def kernel(sup_x, y, assign_index, assign_weight, anchor_links, anchor_weight,
           num_nodes, W1, W2):
    srca = assign_index[0].astype(jnp.int32)
    dsta = assign_index[1].astype(jnp.int32)
    srcb = anchor_links[0].astype(jnp.int32)
    dstb = anchor_links[1].astype(jnp.int32)
    wa = assign_weight
    wb = anchor_weight

    t1, t2 = pl.pallas_call(
        _mm_body,
        grid=(N_NODES // BM,),
        in_specs=[
            pl.BlockSpec((BM, D), lambda i: (i, 0)),
            pl.BlockSpec((BM, D), lambda i: (i, 0)),
            pl.BlockSpec((D, D), lambda i: (0, 0)),
            pl.BlockSpec((D, D), lambda i: (0, 0)),
        ],
        out_specs=[
            pl.BlockSpec((BM, D), lambda i: (i, 0)),
            pl.BlockSpec((BM, D), lambda i: (i, 0)),
        ],
        out_shape=[jax.ShapeDtypeStruct((NPAD, D), jnp.float32)] * 2,
    )(sup_x, y, W1, W2)

    zeros_hbm = jnp.zeros((NPAD, D), jnp.float32)
    partial = _scatter_partials(t1, t2, srca, dsta, wa, srcb, dstb, wb, zeros_hbm)

    z = pl.pallas_call(
        _add_body,
        grid=(N_NODES // BM,),
        in_specs=[
            pl.BlockSpec((BM, D), lambda i: (i, 0)),
            pl.BlockSpec((BM, D), lambda i: (i, 0)),
        ],
        out_specs=pl.BlockSpec((BM, D), lambda i: (i, 0)),
        out_shape=jax.ShapeDtypeStruct((N_NODES, D), jnp.float32),
    )(partial[0], partial[1])
    return z


# speedup vs baseline: 9.6372x; 1.6605x over previous
import jax
import jax.numpy as jnp
from jax import lax
from jax.experimental import pallas as pl
from jax.experimental.pallas import tpu as pltpu
from jax.experimental.pallas import tpu_sc as plsc

D = 128
N_NODES = 10000
N_EDGES = 320000

NC = 2
NS = 16
NW = NC * NS

NPAD = 10240
ROWS_PT = NPAD // NS

CH = 80
NCH = 252
NPAIR = NCH // 2
EPT = NCH * CH
E_TOT = NW * EPT


def _lane_bcast(vec, j):
    idx = jnp.full((16, 1), j, dtype=jnp.int32)
    return lax.gather(
        vec, idx,
        lax.GatherDimensionNumbers(
            offset_dims=(), collapsed_slice_dims=(0,), start_index_map=(0,)),
        (1,),
        mode=lax.GatherScatterMode.PROMISE_IN_BOUNDS)


def _sc_body(tab, srch, dsth, wh, zeros_hbm, out,
             acc, rows0, rows1,
             srcs0, srcs1, srcs2, dsts0, dsts1, dsts2, ws0, ws1, ws2,
             gsem, ssem, isem):
    cid = lax.axis_index("c")
    sid = lax.axis_index("s")
    wid = sid * NC + cid

    rows = (rows0, rows1)
    srcs = (srcs0, srcs1, srcs2)
    dsts = (dsts0, dsts1, dsts2)
    ws = (ws0, ws1, ws2)

    pltpu.sync_copy(zeros_hbm.at[pl.ds(sid * ROWS_PT, ROWS_PT)],
                    acc.at[pl.ds(sid * ROWS_PT, ROWS_PT)])
    plsc.subcore_barrier()

    base = wid * EPT

    def stage_idx(c, s):
        off = base + c * CH
        pltpu.async_copy(srch.at[pl.ds(off, CH)], srcs[s], isem)
        pltpu.async_copy(dsth.at[pl.ds(off, CH)], dsts[s], isem)
        pltpu.async_copy(wh.at[pl.ds(off, CH)], ws[s], isem)

    def wait_idx(s):
        pltpu.make_async_copy(srch.at[pl.ds(0, CH)], srcs[s], isem).wait()
        pltpu.make_async_copy(dsth.at[pl.ds(0, CH)], dsts[s], isem).wait()
        pltpu.make_async_copy(wh.at[pl.ds(0, CH)], ws[s], isem).wait()

    def start_gather(b, s):
        pltpu.async_copy(tab.at[srcs[s]], rows[b], gsem)

    def wait_gather(b, s):
        pltpu.make_async_copy(tab.at[srcs[s]], rows[b], gsem).wait()

    def start_scatter(b, s):
        pltpu.async_copy(rows[b], acc.at[dsts[s]], ssem, add=True)

    def wait_scatter(b, s):
        pltpu.make_async_copy(rows[b], acc.at[dsts[s]], ssem).wait()

    def scale(b, s):
        rbuf = rows[b]
        wref = ws[s]

        def grp(g, carry):
            wv = wref[pl.ds(16 * g, 16)]
            for j in range(16):
                wbc = _lane_bcast(wv, j)
                e = 16 * g + j
                for v in range(8):
                    rbuf[e, pl.ds(16 * v, 16)] = rbuf[e, pl.ds(16 * v, 16)] * wbc
            return carry
        lax.fori_loop(0, CH // 16, grp, 0)

    def chunk(c, b, s, first, guard):
        bn = 1 - b
        sn = (s + 1) % 3
        sp = (s + 2) % 3
        if not first:
            wait_scatter(bn, sp)
        if guard:
            @pl.when(c + 1 < NCH)
            def _():
                wait_idx(sn)

            @pl.when(c + 2 < NCH)
            def _():
                stage_idx(c + 2, sp)

            @pl.when(c + 1 < NCH)
            def _():
                start_gather(bn, sn)
        else:
            wait_idx(sn)
            stage_idx(c + 2, sp)
            start_gather(bn, sn)
        wait_gather(b, s)
        scale(b, s)
        start_scatter(b, s)

    stage_idx(0, 0)
    stage_idx(1, 1)
    wait_idx(0)
    start_gather(0, 0)
    chunk(0, 0, 0, True, False)
    chunk(1, 1, 1, False, False)
    chunk(2, 0, 2, False, False)
    chunk(3, 1, 0, False, False)
    chunk(4, 0, 1, False, False)
    chunk(5, 1, 2, False, False)

    def loop_body(j, carry):
        for r in range(6):
            chunk(6 * j + r, r % 2, r % 3, False, r >= 4)
        return carry

    lax.fori_loop(1, NCH // 6, loop_body, 0)

    wait_scatter(1, (NCH - 1) % 3)

    plsc.subcore_barrier()
    pltpu.sync_copy(acc.at[pl.ds(sid * ROWS_PT, ROWS_PT)],
                    out.at[cid, pl.ds(sid * ROWS_PT, ROWS_PT)])


def _scatter_partials(tab, srch, dsth, wh, zeros_hbm):
    mesh = plsc.VectorSubcoreMesh(core_axis_name="c", subcore_axis_name="s")
    return pl.kernel(
        _sc_body,
        mesh=mesh,
        out_type=jax.ShapeDtypeStruct((NC, NPAD, D), jnp.float32),
        scratch_types=[
            pltpu.VMEM_SHARED((NPAD, D), jnp.float32),
            pltpu.VMEM((CH, D), jnp.float32),
            pltpu.VMEM((CH, D), jnp.float32),
            pltpu.VMEM((CH,), jnp.int32),
            pltpu.VMEM((CH,), jnp.int32),
            pltpu.VMEM((CH,), jnp.int32),
            pltpu.VMEM((CH,), jnp.int32),
            pltpu.VMEM((CH,), jnp.int32),
            pltpu.VMEM((CH,), jnp.int32),
            pltpu.VMEM((CH,), jnp.float32),
            pltpu.VMEM((CH,), jnp.float32),
            pltpu.VMEM((CH,), jnp.float32),
            pltpu.SemaphoreType.DMA,
            pltpu.SemaphoreType.DMA,
            pltpu.SemaphoreType.DMA,
        ],
    )(tab, srch, dsth, wh, zeros_hbm)


BM = 2048


def _mm_body(x_ref, y_ref, w1_ref, w2_ref, o_ref):
    dn = (((1,), (1,)), ((), ()))
    s = pl.program_id(0)

    @pl.when(s == 0)
    def _():
        o_ref[...] = lax.dot_general(x_ref[...], w1_ref[...], dn,
                                     preferred_element_type=jnp.float32)[None]

    @pl.when(s == 1)
    def _():
        o_ref[...] = lax.dot_general(y_ref[...], w2_ref[...], dn,
                                     preferred_element_type=jnp.float32)[None]


def _add_body(a_ref, b_ref, o_ref):
    o_ref[...] = a_ref[...] + b_ref[...]


def kernel(sup_x, y, assign_index, assign_weight, anchor_links, anchor_weight,
           num_nodes, W1, W2):
    srca = assign_index[0].astype(jnp.int32)
    dsta = assign_index[1].astype(jnp.int32)
    srcb = anchor_links[0].astype(jnp.int32) + NPAD
    dstb = anchor_links[1].astype(jnp.int32)

    npad_e = E_TOT - 2 * N_EDGES
    pad_idx = (jnp.arange(npad_e, dtype=jnp.int32) % N_NODES)
    src_all = jnp.concatenate([srca, srcb, pad_idx])
    dst_all = jnp.concatenate([dsta, dstb, pad_idx])
    w_all = jnp.concatenate(
        [assign_weight, anchor_weight, jnp.zeros((npad_e,), jnp.float32)])

    npb = NPAD // BM
    tab = pl.pallas_call(
        _mm_body,
        grid=(2, npb),
        in_specs=[
            pl.BlockSpec((BM, D), lambda s, i: (i, 0)),
            pl.BlockSpec((BM, D), lambda s, i: (i, 0)),
            pl.BlockSpec((D, D), lambda s, i: (0, 0)),
            pl.BlockSpec((D, D), lambda s, i: (0, 0)),
        ],
        out_specs=pl.BlockSpec((1, BM, D), lambda s, i: (s, i, 0)),
        out_shape=jax.ShapeDtypeStruct((2, NPAD, D), jnp.float32),
    )(jnp.pad(sup_x, ((0, NPAD - N_NODES), (0, 0))),
      jnp.pad(y, ((0, NPAD - N_NODES), (0, 0))), W1, W2)

    zeros_hbm = jnp.zeros((NPAD, D), jnp.float32)
    partial = _scatter_partials(tab.reshape(2 * NPAD, D),
                                src_all, dst_all, w_all, zeros_hbm)

    z = pl.pallas_call(
        _add_body,
        grid=(N_NODES // 2000,),
        in_specs=[
            pl.BlockSpec((2000, D), lambda i: (i, 0)),
            pl.BlockSpec((2000, D), lambda i: (i, 0)),
        ],
        out_specs=pl.BlockSpec((2000, D), lambda i: (i, 0)),
        out_shape=jax.ShapeDtypeStruct((N_NODES, D), jnp.float32),
    )(partial[0, :N_NODES], partial[1, :N_NODES])
    return z


# speedup vs baseline: 11.3881x; 1.1817x over previous
import jax
import jax.numpy as jnp
from jax import lax
from jax.experimental import pallas as pl
from jax.experimental.pallas import tpu as pltpu
from jax.experimental.pallas import tpu_sc as plsc

D = 128
N_NODES = 10000
N_EDGES = 320000

NC = 2
NS = 16
NW = NC * NS

NPAD = 10240
ROWS_PT = NPAD // NS

CH = 80
NCH = 252
EPT = NCH * CH
E_TOT = NW * EPT


def _lane_bcast(vec, j):
    idx = jnp.full((16, 1), j, dtype=jnp.int32)
    return lax.gather(
        vec, idx,
        lax.GatherDimensionNumbers(
            offset_dims=(), collapsed_slice_dims=(0,), start_index_map=(0,)),
        (1,),
        mode=lax.GatherScatterMode.PROMISE_IN_BOUNDS)


def _sc_body(tab, srch, dsth, wh, zeros_hbm, out,
             acc, rows0, rows1, rows2, rows3,
             srcs0, srcs1, srcs2, srcs3, dsts0, dsts1, dsts2, dsts3,
             ws0, ws1, ws2, ws3, dsc0, dsc1, g0, g1, ssem, isem):
    cid = lax.axis_index("c")
    sid = lax.axis_index("s")
    wid = sid * NC + cid

    rows = (rows0, rows1, rows2, rows3)
    srcs = (srcs0, srcs1, srcs2, srcs3)
    dsts = (dsts0, dsts1, dsts2, dsts3)
    ws = (ws0, ws1, ws2, ws3)
    dsc = (dsc0, dsc1)
    gsem = (g0, g1)

    pltpu.sync_copy(zeros_hbm.at[pl.ds(sid * ROWS_PT, ROWS_PT)],
                    acc.at[pl.ds(sid * ROWS_PT, ROWS_PT)])
    plsc.subcore_barrier()

    base = wid * EPT

    def stage_idx(c, s):
        off = base + c * CH
        pltpu.async_copy(srch.at[pl.ds(off, CH)], srcs[s], isem)
        pltpu.async_copy(dsth.at[pl.ds(off, CH)], dsts[s], isem)
        pltpu.async_copy(wh.at[pl.ds(off, CH)], ws[s], isem)

    def wait_idx(s):
        pltpu.make_async_copy(srch.at[pl.ds(0, CH)], srcs[s], isem).wait()
        pltpu.make_async_copy(dsth.at[pl.ds(0, CH)], dsts[s], isem).wait()
        pltpu.make_async_copy(wh.at[pl.ds(0, CH)], ws[s], isem).wait()

    def start_gather(l):
        pltpu.async_copy(tab.at[srcs[l]], rows[l], gsem[l % 2])

    def wait_gather(l):
        pltpu.make_async_copy(tab.at[srcs[l]], rows[l], gsem[l % 2]).wait()

    def start_scatter(l, p):
        for k in range(CH // 16):
            dsc[p][pl.ds(16 * k, 16)] = dsts[l][pl.ds(16 * k, 16)]
        pltpu.async_copy(rows[l], acc.at[dsc[p]], ssem, add=True)

    def wait_scatter(l, p):
        pltpu.make_async_copy(rows[l], acc.at[dsc[p]], ssem).wait()

    def scale(b, s):
        rbuf = rows[b]
        wref = ws[s]

        def grp(g, carry):
            wv = wref[pl.ds(16 * g, 16)]
            for j in range(16):
                wbc = _lane_bcast(wv, j)
                e = 16 * g + j
                for v in range(8):
                    rbuf[e, pl.ds(16 * v, 16)] = rbuf[e, pl.ds(16 * v, 16)] * wbc
            return carry
        lax.fori_loop(0, CH // 16, grp, 0)

    def chunk(c, r, first, guard):
        l = r
        l2 = (r + 2) % 4
        l3 = (r + 3) % 4
        wait_gather(l)
        if guard:
            @pl.when(c + 2 < NCH)
            def _():
                wait_idx(l2)

            @pl.when(c + 3 < NCH)
            def _():
                stage_idx(c + 3, l3)

            @pl.when(c + 2 < NCH)
            def _():
                start_gather(l2)
        else:
            wait_idx(l2)
            stage_idx(c + 3, l3)
            start_gather(l2)
        scale(l, l)
        if not first:
            wait_scatter(l3, (r + 1) % 2)
        start_scatter(l, r % 2)

    stage_idx(0, 0)
    wait_idx(0)
    start_gather(0)
    stage_idx(1, 1)
    wait_idx(1)
    start_gather(1)
    stage_idx(2, 2)
    chunk(0, 0, True, False)
    chunk(1, 1, False, False)
    chunk(2, 2, False, False)
    chunk(3, 3, False, False)

    def loop_body(j, carry):
        for r in range(4):
            chunk(4 * j + r, r, False, True)
        return carry

    lax.fori_loop(1, NCH // 4, loop_body, 0)

    wait_scatter((NCH - 1) % 4, (NCH - 1) % 2)

    plsc.subcore_barrier()
    pltpu.sync_copy(acc.at[pl.ds(sid * ROWS_PT, ROWS_PT)],
                    out.at[cid, pl.ds(sid * ROWS_PT, ROWS_PT)])


def _scatter_partials(tab, srch, dsth, wh, zeros_hbm):
    mesh = plsc.VectorSubcoreMesh(core_axis_name="c", subcore_axis_name="s")
    return pl.kernel(
        _sc_body,
        mesh=mesh,
        out_type=jax.ShapeDtypeStruct((NC, NPAD, D), jnp.float32),
        scratch_types=[
            pltpu.VMEM_SHARED((NPAD, D), jnp.float32),
            pltpu.VMEM((CH, D), jnp.float32),
            pltpu.VMEM((CH, D), jnp.float32),
            pltpu.VMEM((CH, D), jnp.float32),
            pltpu.VMEM((CH, D), jnp.float32),
            pltpu.VMEM((CH,), jnp.int32),
            pltpu.VMEM((CH,), jnp.int32),
            pltpu.VMEM((CH,), jnp.int32),
            pltpu.VMEM((CH,), jnp.int32),
            pltpu.VMEM((CH,), jnp.int32),
            pltpu.VMEM((CH,), jnp.int32),
            pltpu.VMEM((CH,), jnp.int32),
            pltpu.VMEM((CH,), jnp.int32),
            pltpu.VMEM((CH,), jnp.float32),
            pltpu.VMEM((CH,), jnp.float32),
            pltpu.VMEM((CH,), jnp.float32),
            pltpu.VMEM((CH,), jnp.float32),
            pltpu.VMEM((CH,), jnp.int32),
            pltpu.VMEM((CH,), jnp.int32),
            pltpu.SemaphoreType.DMA,
            pltpu.SemaphoreType.DMA,
            pltpu.SemaphoreType.DMA,
            pltpu.SemaphoreType.DMA,
        ],
    )(tab, srch, dsth, wh, zeros_hbm)


BM = 2048


def _mm_body(x_ref, y_ref, w1_ref, w2_ref, o_ref):
    dn = (((1,), (1,)), ((), ()))
    s = pl.program_id(0)

    @pl.when(s == 0)
    def _():
        o_ref[...] = lax.dot_general(x_ref[...], w1_ref[...], dn,
                                     preferred_element_type=jnp.float32)[None]

    @pl.when(s == 1)
    def _():
        o_ref[...] = lax.dot_general(y_ref[...], w2_ref[...], dn,
                                     preferred_element_type=jnp.float32)[None]


def _add_body(a_ref, b_ref, o_ref):
    o_ref[...] = a_ref[...] + b_ref[...]


def kernel(sup_x, y, assign_index, assign_weight, anchor_links, anchor_weight,
           num_nodes, W1, W2):
    srca = assign_index[0].astype(jnp.int32)
    dsta = assign_index[1].astype(jnp.int32)
    srcb = anchor_links[0].astype(jnp.int32) + NPAD
    dstb = anchor_links[1].astype(jnp.int32)

    npad_e = E_TOT - 2 * N_EDGES
    pad_idx = (jnp.arange(npad_e, dtype=jnp.int32) % N_NODES)
    src_all = jnp.concatenate([srca, srcb, pad_idx])
    dst_all = jnp.concatenate([dsta, dstb, pad_idx])
    w_all = jnp.concatenate(
        [assign_weight, anchor_weight, jnp.zeros((npad_e,), jnp.float32)])

    npb = NPAD // BM
    tab = pl.pallas_call(
        _mm_body,
        grid=(2, npb),
        in_specs=[
            pl.BlockSpec((BM, D), lambda s, i: (i, 0)),
            pl.BlockSpec((BM, D), lambda s, i: (i, 0)),
            pl.BlockSpec((D, D), lambda s, i: (0, 0)),
            pl.BlockSpec((D, D), lambda s, i: (0, 0)),
        ],
        out_specs=pl.BlockSpec((1, BM, D), lambda s, i: (s, i, 0)),
        out_shape=jax.ShapeDtypeStruct((2, NPAD, D), jnp.float32),
    )(jnp.pad(sup_x, ((0, NPAD - N_NODES), (0, 0))),
      jnp.pad(y, ((0, NPAD - N_NODES), (0, 0))), W1, W2)

    zeros_hbm = jnp.zeros((NPAD, D), jnp.float32)
    partial = _scatter_partials(tab.reshape(2 * NPAD, D),
                                src_all, dst_all, w_all, zeros_hbm)

    z = pl.pallas_call(
        _add_body,
        grid=(N_NODES // 2000,),
        in_specs=[
            pl.BlockSpec((2000, D), lambda i: (i, 0)),
            pl.BlockSpec((2000, D), lambda i: (i, 0)),
        ],
        out_specs=pl.BlockSpec((2000, D), lambda i: (i, 0)),
        out_shape=jax.ShapeDtypeStruct((N_NODES, D), jnp.float32),
    )(partial[0, :N_NODES], partial[1, :N_NODES])
    return z
